# baseline (speedup 1.0000x reference)
import functools

import jax
import jax.numpy as jnp
from jax import lax
from jax.experimental import pallas as pl
from jax.experimental.pallas import tpu as pltpu
from jax.experimental.pallas import tpu_sc as plsc

N_CONCEPTS = 1000000
NLAYERS = 2
HIDDEN = 64
BATCH = 16384

_info = plsc.get_sparse_core_info()
NC = _info.num_cores
NS = _info.num_subcores
NW = NC * NS
B_PER_W = BATCH // NW
CHUNK = 128
NCHUNK = B_PER_W // CHUNK


def _sc_body(x_ref, c_ref, h_ref, c_out, h_out, idx_v, rows_v, sem):
    wid = lax.axis_index("s") * NC + lax.axis_index("c")
    pltpu.sync_copy(x_ref.at[wid], idx_v)
    for t_ref, o_ref in ((c_ref, c_out), (h_ref, h_out)):
        for j in range(NCHUNK):
            pltpu.async_copy(t_ref.at[idx_v.at[j]], rows_v, sem).wait()
            base = wid * B_PER_W + j * CHUNK
            for l in range(NLAYERS):
                pltpu.sync_copy(
                    rows_v.at[:, pl.ds(l * HIDDEN, HIDDEN)],
                    o_ref.at[l, pl.ds(base, CHUNK), :],
                )


@functools.partial(jax.jit, static_argnums=())
def kernel(x, c_table, h_table):
    x3 = x.reshape(NW, NCHUNK, CHUNK)
    out_sds = jax.ShapeDtypeStruct((NLAYERS, BATCH, HIDDEN), jnp.float32)
    run = pl.kernel(
        _sc_body,
        out_type=(out_sds, out_sds),
        mesh=plsc.VectorSubcoreMesh(core_axis_name="c", subcore_axis_name="s"),
        compiler_params=pltpu.CompilerParams(use_tc_tiling_on_sc=False),
        scratch_types=[
            pltpu.VMEM((NCHUNK, CHUNK), jnp.int32),
            pltpu.VMEM((CHUNK, NLAYERS * HIDDEN), jnp.float32),
            pltpu.SemaphoreType.DMA,
        ],
    )
    c_init, h_init = run(x3, c_table, h_table)
    return (c_init, h_init)


# speedup vs baseline: 1.1165x; 1.1165x over previous
import functools

import jax
import jax.numpy as jnp
from jax import lax
from jax.experimental import pallas as pl
from jax.experimental.pallas import tpu as pltpu
from jax.experimental.pallas import tpu_sc as plsc

N_CONCEPTS = 1000000
NLAYERS = 2
HIDDEN = 64
BATCH = 16384

_info = plsc.get_sparse_core_info()
NC = _info.num_cores
NS = _info.num_subcores
NW = NC * NS
B_PER_W = BATCH // NW
CHUNK = 128
NCHUNK = B_PER_W // CHUNK


NBUF = 6
NGATHER = 2 * NCHUNK


def _sc_body(x_ref, c_ref, h_ref, c_out, h_out, idx_v, bufs, gsems, wsems):
    wid = lax.axis_index("s") * NC + lax.axis_index("c")
    pltpu.sync_copy(x_ref.at[wid], idx_v)

    tables = (c_ref, h_ref)
    outs = (c_out, h_out)

    def start_gather(g):
        t, j = divmod(g, NCHUNK)
        b = g % NBUF
        return pltpu.async_copy(
            tables[t].at[idx_v.at[j]], bufs.at[b], gsems.at[b]
        )

    gh = [start_gather(g) for g in range(NBUF)]
    gh += [None] * (NGATHER - NBUF)
    wh = [None] * NBUF

    for g in range(NGATHER):
        t, j = divmod(g, NCHUNK)
        b = g % NBUF
        gh[g].wait()
        base = wid * B_PER_W + j * CHUNK
        wh[b] = [
            pltpu.async_copy(
                bufs.at[b, :, pl.ds(l * HIDDEN, HIDDEN)],
                outs[t].at[l, pl.ds(base, CHUNK), :],
                wsems.at[b],
            )
            for l in range(NLAYERS)
        ]
        ng = g + 3
        if NBUF <= ng < NGATHER:
            nb = ng % NBUF
            for w in wh[nb]:
                w.wait()
            gh[ng] = start_gather(ng)
    for g in range(NGATHER - NBUF, NGATHER):
        for w in wh[g % NBUF]:
            w.wait()


@functools.partial(jax.jit, static_argnums=())
def kernel(x, c_table, h_table):
    x3 = x.reshape(NW, NCHUNK, CHUNK)
    out_sds = jax.ShapeDtypeStruct((NLAYERS, BATCH, HIDDEN), jnp.float32)
    run = pl.kernel(
        _sc_body,
        out_type=(out_sds, out_sds),
        mesh=plsc.VectorSubcoreMesh(core_axis_name="c", subcore_axis_name="s"),
        compiler_params=pltpu.CompilerParams(use_tc_tiling_on_sc=False),
        scratch_types=[
            pltpu.VMEM((NCHUNK, CHUNK), jnp.int32),
            pltpu.VMEM((NBUF, CHUNK, NLAYERS * HIDDEN), jnp.float32),
            pltpu.SemaphoreType.DMA((NBUF,)),
            pltpu.SemaphoreType.DMA((NBUF,)),
        ],
    )
    c_init, h_init = run(x3, c_table, h_table)
    return (c_init, h_init)


# speedup vs baseline: 1.5578x; 1.3952x over previous
import functools

import jax
import jax.numpy as jnp
from jax import lax
from jax.experimental import pallas as pl
from jax.experimental.pallas import tpu as pltpu
from jax.experimental.pallas import tpu_sc as plsc

N_CONCEPTS = 1000000
NLAYERS = 2
HIDDEN = 64
BATCH = 16384

_info = plsc.get_sparse_core_info()
NC = _info.num_cores
NS = _info.num_subcores
NW = NC * NS
B_PER_W = BATCH // NW
CHUNK = 128
NCHUNK = B_PER_W // CHUNK


NBUF = 6
NGATHER = 2 * NCHUNK


def _sc_body(x_ref, c_ref, h_ref, c_out, h_out, idx_v, bufs, gsems, wsems):
    wid = lax.axis_index("s") * NC + lax.axis_index("c")
    pltpu.sync_copy(x_ref.at[wid], idx_v)

    tables = (c_ref, h_ref)
    outs = (c_out, h_out)

    def start_gather(g):
        t, j = divmod(g, NCHUNK)
        b = g % NBUF
        return pltpu.async_copy(
            tables[t].at[idx_v.at[j]], bufs.at[b], gsems.at[b]
        )

    gh = [start_gather(g) for g in range(NBUF)]
    gh += [None] * (NGATHER - NBUF)
    wh = [None] * NBUF

    for g in range(NGATHER):
        t, j = divmod(g, NCHUNK)
        b = g % NBUF
        gh[g].wait()
        base = wid * B_PER_W + j * CHUNK
        wh[b] = pltpu.async_copy(
            bufs.at[b], outs[t].at[pl.ds(base, CHUNK), :], wsems.at[b]
        )
        ng = g + 3
        if NBUF <= ng < NGATHER:
            wh[ng % NBUF].wait()
            gh[ng] = start_gather(ng)
    for g in range(NGATHER - NBUF, NGATHER):
        wh[g % NBUF].wait()


@functools.partial(jax.jit, static_argnums=())
def kernel(x, c_table, h_table):
    x3 = x.reshape(NW, NCHUNK, CHUNK)
    out_sds = jax.ShapeDtypeStruct((BATCH, NLAYERS * HIDDEN), jnp.float32)
    run = pl.kernel(
        _sc_body,
        out_type=(out_sds, out_sds),
        mesh=plsc.VectorSubcoreMesh(core_axis_name="c", subcore_axis_name="s"),
        compiler_params=pltpu.CompilerParams(use_tc_tiling_on_sc=False),
        scratch_types=[
            pltpu.VMEM((NCHUNK, CHUNK), jnp.int32),
            pltpu.VMEM((NBUF, CHUNK, NLAYERS * HIDDEN), jnp.float32),
            pltpu.SemaphoreType.DMA((NBUF,)),
            pltpu.SemaphoreType.DMA((NBUF,)),
        ],
    )
    c_rows, h_rows = run(x3, c_table, h_table)
    c_init = c_rows.reshape(BATCH, NLAYERS, HIDDEN).transpose(1, 0, 2)
    h_init = h_rows.reshape(BATCH, NLAYERS, HIDDEN).transpose(1, 0, 2)
    return (c_init, h_init)


# speedup vs baseline: 1.6605x; 1.0659x over previous
import functools

import jax
import jax.numpy as jnp
from jax import lax
from jax.experimental import pallas as pl
from jax.experimental.pallas import tpu as pltpu
from jax.experimental.pallas import tpu_sc as plsc

N_CONCEPTS = 1000000
NLAYERS = 2
HIDDEN = 64
BATCH = 16384

_info = plsc.get_sparse_core_info()
NC = _info.num_cores
NS = _info.num_subcores
NW = NC * NS
B_PER_W = BATCH // NW
CHUNK = 128
NCHUNK = B_PER_W // CHUNK


NBUF = 6
NGATHER = 2 * NCHUNK


def _sc_body(x_ref, c_ref, h_ref, c_out, h_out, idx_v, bufs, gsems, wsems):
    wid = lax.axis_index("s") * NC + lax.axis_index("c")
    row0 = pl.multiple_of((wid // 2) * (2 * NCHUNK), 8)
    pltpu.sync_copy(x_ref.at[pl.ds(row0, 2 * NCHUNK), :], idx_v)

    tables = (c_ref, h_ref)
    outs = (c_out, h_out)

    def start_gather(g):
        t, j = divmod(g, NCHUNK)
        b = g % NBUF
        return pltpu.async_copy(
            tables[t].at[idx_v.at[(wid % 2) * NCHUNK + j]],
            bufs.at[b],
            gsems.at[b],
        )

    gh = [start_gather(g) for g in range(NBUF)]
    gh += [None] * (NGATHER - NBUF)
    wh = [None] * NBUF

    for g in range(NGATHER):
        t, j = divmod(g, NCHUNK)
        b = g % NBUF
        gh[g].wait()
        base = wid * B_PER_W + j * CHUNK
        wh[b] = pltpu.async_copy(
            bufs.at[b], outs[t].at[pl.ds(base, CHUNK), :], wsems.at[b]
        )
        ng = g + 3
        if NBUF <= ng < NGATHER:
            wh[ng % NBUF].wait()
            gh[ng] = start_gather(ng)
    for g in range(NGATHER - NBUF, NGATHER):
        wh[g % NBUF].wait()


@functools.partial(jax.jit, static_argnums=())
def kernel(x, c_table, h_table):
    x3 = x.reshape(BATCH // CHUNK, CHUNK)
    out_sds = jax.ShapeDtypeStruct((BATCH, NLAYERS * HIDDEN), jnp.float32)
    run = pl.kernel(
        _sc_body,
        out_type=(out_sds, out_sds),
        mesh=plsc.VectorSubcoreMesh(core_axis_name="c", subcore_axis_name="s"),
        scratch_types=[
            pltpu.VMEM((2 * NCHUNK, CHUNK), jnp.int32),
            pltpu.VMEM((NBUF, CHUNK, NLAYERS * HIDDEN), jnp.float32),
            pltpu.SemaphoreType.DMA((NBUF,)),
            pltpu.SemaphoreType.DMA((NBUF,)),
        ],
    )
    c_rows, h_rows = run(x3, c_table, h_table)
    c_init = c_rows.reshape(BATCH, NLAYERS, HIDDEN).transpose(1, 0, 2)
    h_init = h_rows.reshape(BATCH, NLAYERS, HIDDEN).transpose(1, 0, 2)
    return (c_init, h_init)
